# baseline (speedup 1.0000x reference)
import functools

import jax
import jax.numpy as jnp
from jax import lax
from jax.experimental import pallas as pl
from jax.experimental.pallas import tpu as pltpu
from jax.experimental.pallas import tpu_sc as plsc

N_NODES = 50000
N_EDGES = 800000
DIM = 100
DAGG = 50
DCLS = 30

ROW_BLK = 1024
NP = 50176
GRID = NP // ROW_BLK

NB = 4
CB = 25
CBP = 32

ND = 51200
TILE_ROWS = ND // 16
DUMMY_DST = 50000

EK = 128
TILE_EDGES = 52224
EP = TILE_EDGES * 16
N_BATCH = TILE_EDGES // EK
PER_TILE = N_EDGES // 16
SLACK = TILE_EDGES - PER_TILE


def _write_blocked(pre_ref, fp):
    r = fp.shape[0]
    pad = jnp.zeros((r, CBP - CB), jnp.float32)
    for j in range(NB):
        pre_ref[j] = jnp.concatenate([fp[:, j * CB:(j + 1) * CB], pad], axis=1)


def _init_body(feat_ref, wi_ref, bi_ref, w1_ref, b1_ref, w2_ref, b2_ref,
               h_ref, pre_ref):
    h = jnp.dot(feat_ref[...], wi_ref[...],
                preferred_element_type=jnp.float32) + bi_ref[...]
    h_ref[...] = h
    a = jnp.maximum(jnp.dot(h, w1_ref[...],
                            preferred_element_type=jnp.float32) + b1_ref[...], 0.0)
    fp = jnp.dot(a, w2_ref[...], preferred_element_type=jnp.float32) + b2_ref[...]
    _write_blocked(pre_ref, fp)


def _gru_common(msg_ref, h_ref, wir, wiz, win, whr, whz, whn,
                bir, biz, bin_, bhr, bhz, bhn):
    x = jnp.concatenate([msg_ref[j, :, :CB] for j in range(NB)], axis=1)
    h = h_ref[...]
    d = functools.partial(jnp.dot, preferred_element_type=jnp.float32)
    r = jax.nn.sigmoid(d(x, wir[...]) + bir[...] + d(h, whr[...]) + bhr[...])
    z = jax.nn.sigmoid(d(x, wiz[...]) + biz[...] + d(h, whz[...]) + bhz[...])
    n = jnp.tanh(d(x, win[...]) + bin_[...] + r * (d(h, whn[...]) + bhn[...]))
    return (1.0 - z) * n + z * h


def _gru_mlp_body(msg_ref, h_ref, wir, wiz, win, whr, whz, whn,
                  bir, biz, bin_, bhr, bhz, bhn,
                  w1_ref, b1_ref, w2_ref, b2_ref, hout_ref, pre_ref):
    hn = _gru_common(msg_ref, h_ref, wir, wiz, win, whr, whz, whn,
                     bir, biz, bin_, bhr, bhz, bhn)
    hout_ref[...] = hn
    a = jnp.maximum(jnp.dot(hn, w1_ref[...],
                            preferred_element_type=jnp.float32) + b1_ref[...], 0.0)
    fp = jnp.dot(a, w2_ref[...], preferred_element_type=jnp.float32) + b2_ref[...]
    _write_blocked(pre_ref, fp)


def _gru_cls_body(msg_ref, h_ref, wir, wiz, win, whr, whz, whn,
                  bir, biz, bin_, bhr, bhz, bhn,
                  w1_ref, b1_ref, w2_ref, b2_ref, out_ref):
    hn = _gru_common(msg_ref, h_ref, wir, wiz, win, whr, whz, whn,
                     bir, biz, bin_, bhr, bhz, bhn)
    a = jnp.maximum(jnp.dot(hn, w1_ref[...],
                            preferred_element_type=jnp.float32) + b1_ref[...], 0.0)
    out_ref[...] = jnp.dot(a, w2_ref[...],
                           preferred_element_type=jnp.float32) + b2_ref[...]


def _row_spec(cols):
    return pl.BlockSpec((ROW_BLK, cols), lambda i: (i, 0))


def _full_spec(shape):
    nd = len(shape)
    return pl.BlockSpec(shape, lambda i, _n=nd: (0,) * _n)


def _msg_spec():
    return pl.BlockSpec((NB, ROW_BLK, CBP), lambda i: (0, i, 0))


def _pre_out_spec():
    return pl.BlockSpec((NB, ROW_BLK, CBP), lambda i: (0, i, 0))


def _segsum_body(pre_hbm, gidx_hbm, sidx_hbm, out_hbm,
                 gidx_v0, gidx_v1, sidx_v0, sidx_v1, rows_v0, rows_v1,
                 acc_sh, sem0, sem1):
    c = lax.axis_index("c")
    s = lax.axis_index("s")
    row_base = s * TILE_ROWS
    edge_base = s * TILE_EDGES
    gidx = (gidx_v0, gidx_v1)
    sidx = (sidx_v0, sidx_v1)
    rows = (rows_v0, rows_v1)
    sems = (sem0, sem1)

    for p in range(2):
        blk = c * 2 + p

        def _zrow(i, _):
            rows_v0[i // 2, pl.ds((i % 2) * 16, 16)] = jnp.zeros((16,), jnp.float32)
            return _

        lax.fori_loop(0, EK * 2, _zrow, None)

        def _zacc(q, _):
            pltpu.sync_copy(rows_v0, acc_sh.at[pl.ds(row_base + q * EK, EK)])
            return _

        lax.fori_loop(0, TILE_ROWS // EK, _zacc, None)
        plsc.subcore_barrier()

        pltpu.sync_copy(gidx_hbm.at[pl.ds(edge_base, EK)], gidx[0])
        pltpu.sync_copy(sidx_hbm.at[pl.ds(edge_base, EK)], sidx[0])

        def _pair(g, _):
            for b in range(2):
                i = 2 * g + b
                oth = 1 - b
                hdl = pltpu.async_copy(
                    pre_hbm.at[blk].at[gidx[b]], rows[b], sems[b])

                @pl.when(i > 0)
                def _scatter_prev():
                    pltpu.sync_copy(rows[oth], acc_sh.at[sidx[oth]], add=True)

                @pl.when(i + 1 < N_BATCH)
                def _load_next():
                    pos = edge_base + (i + 1) * EK
                    pltpu.sync_copy(gidx_hbm.at[pl.ds(pos, EK)], gidx[oth])
                    pltpu.sync_copy(sidx_hbm.at[pl.ds(pos, EK)], sidx[oth])

                hdl.wait()
            return _

        lax.fori_loop(0, N_BATCH // 2, _pair, None)
        pltpu.sync_copy(rows[1], acc_sh.at[sidx[1]], add=True)
        plsc.subcore_barrier()

        pltpu.sync_copy(acc_sh.at[pl.ds(row_base, TILE_ROWS)],
                        out_hbm.at[blk].at[pl.ds(row_base, TILE_ROWS)])


_SEGSUM_CACHE = []


def _segsum_sc(pre, gidx, sidx):
    if not _SEGSUM_CACHE:
        mesh = plsc.VectorSubcoreMesh(core_axis_name="c", subcore_axis_name="s")
        _SEGSUM_CACHE.append(pl.kernel(
            _segsum_body,
            out_type=jax.ShapeDtypeStruct((NB, ND, CBP), jnp.float32),
            mesh=mesh,
            scratch_types=[
                pltpu.VMEM((EK,), jnp.int32),
                pltpu.VMEM((EK,), jnp.int32),
                pltpu.VMEM((EK,), jnp.int32),
                pltpu.VMEM((EK,), jnp.int32),
                pltpu.VMEM((EK, CBP), jnp.float32),
                pltpu.VMEM((EK, CBP), jnp.float32),
                pltpu.VMEM_SHARED((ND, CBP), jnp.float32),
                pltpu.SemaphoreType.DMA,
                pltpu.SemaphoreType.DMA,
            ],
            compiler_params=pltpu.CompilerParams(use_tc_tiling_on_sc=False),
        ))
    return _SEGSUM_CACHE[0](pre, gidx, sidx)


def _split3(w):
    return w[:, :DIM], w[:, DIM:2 * DIM], w[:, 2 * DIM:]


def _sorted_layout(scatter_idx, gather_idx):
    order = jnp.argsort(scatter_idx, stable=True)
    ss = scatter_idx[order]
    gs = gather_idx[order]
    seg_start = jnp.searchsorted(ss, ss, side="left").astype(jnp.int32)
    pos = jnp.arange(N_EDGES, dtype=jnp.int32) + SLACK * (seg_start // PER_TILE)
    g = jnp.zeros((EP,), jnp.int32).at[pos].set(gs)
    s = jnp.full((EP,), DUMMY_DST, jnp.int32).at[pos].set(ss)
    return g, s


def kernel(features, edge_index, W_init, b_init, fm_W1, fm_b1, fm_W2, fm_b2,
           bm_W1, bm_b1, bm_W2, bm_b2, fg_Wih, fg_Whh, fg_bih, fg_bhh,
           bg_Wih, bg_Whh, bg_bih, bg_bhh, cl_W1, cl_b1, cl_W2, cl_b2):
    f32 = jnp.float32

    feat_p = jnp.concatenate(
        [features, jnp.zeros((NP - N_NODES, 4), f32)], axis=0)

    dst = edge_index[0]
    src = edge_index[1]
    src_g, dst_s = _sorted_layout(dst, src)
    dst_g, src_s = _sorted_layout(src, dst)

    fir, fiz, fin = _split3(fg_Wih)
    fhr, fhz, fhn = _split3(fg_Whh)
    bir_, biz_, bin_ = _split3(bg_Wih)
    bhr_, bhz_, bhn_ = _split3(bg_Whh)

    def b2d(b):
        return b.reshape(1, -1)

    fb = [b2d(x) for x in jnp.split(fg_bih, 3)] + [b2d(x) for x in jnp.split(fg_bhh, 3)]
    bb = [b2d(x) for x in jnp.split(bg_bih, 3)] + [b2d(x) for x in jnp.split(bg_bhh, 3)]

    h_shape = jax.ShapeDtypeStruct((NP, DIM), f32)
    pre_shape = jax.ShapeDtypeStruct((NB, NP, CBP), f32)

    h, fpre = pl.pallas_call(
        _init_body,
        grid=(GRID,),
        in_specs=[
            _row_spec(4),
            _full_spec((4, DIM)), _full_spec((1, DIM)),
            _full_spec((DIM, DAGG)), _full_spec((1, DAGG)),
            _full_spec((DAGG, DIM)), _full_spec((1, DIM)),
        ],
        out_specs=[_row_spec(DIM), _pre_out_spec()],
        out_shape=[h_shape, pre_shape],
    )(feat_p, W_init, b2d(b_init), fm_W1, b2d(fm_b1), fm_W2, b2d(fm_b2))

    gru_mlp_specs = dict(
        grid=(GRID,),
        in_specs=[
            _msg_spec(), _row_spec(DIM),
            _full_spec((DIM, DIM)), _full_spec((DIM, DIM)), _full_spec((DIM, DIM)),
            _full_spec((DIM, DIM)), _full_spec((DIM, DIM)), _full_spec((DIM, DIM)),
            _full_spec((1, DIM)), _full_spec((1, DIM)), _full_spec((1, DIM)),
            _full_spec((1, DIM)), _full_spec((1, DIM)), _full_spec((1, DIM)),
            _full_spec((DIM, DAGG)), _full_spec((1, DAGG)),
            _full_spec((DAGG, DIM)), _full_spec((1, DIM)),
        ],
    )

    gru_mlp = pl.pallas_call(
        _gru_mlp_body,
        out_specs=[_row_spec(DIM), _pre_out_spec()],
        out_shape=[h_shape, pre_shape],
        **gru_mlp_specs,
    )

    cls_specs = dict(gru_mlp_specs)
    cls_specs["in_specs"] = cls_specs["in_specs"][:-4] + [
        _full_spec((DIM, DCLS)), _full_spec((1, DCLS)),
        _full_spec((DCLS, 1)), _full_spec((1, 1)),
    ]
    gru_cls = pl.pallas_call(
        _gru_cls_body,
        out_specs=[_row_spec(1)],
        out_shape=[jax.ShapeDtypeStruct((NP, 1), f32)],
        **cls_specs,
    )

    for rnd in range(4):
        fmsg = _segsum_sc(fpre, src_g, dst_s)
        h, bpre = gru_mlp(
            fmsg, h, fir, fiz, fin, fhr, fhz, fhn, *fb,
            bm_W1, b2d(bm_b1), bm_W2, b2d(bm_b2))
        bmsg = _segsum_sc(bpre, dst_g, src_s)
        if rnd < 3:
            h, fpre = gru_mlp(
                bmsg, h, bir_, biz_, bin_, bhr_, bhz_, bhn_, *bb,
                fm_W1, b2d(fm_b1), fm_W2, b2d(fm_b2))
        else:
            (out,) = gru_cls(
                bmsg, h, bir_, biz_, bin_, bhr_, bhz_, bhn_, *bb,
                cl_W1, b2d(cl_b1), cl_W2, b2d(cl_b2))

    return out[:N_NODES]


# speedup vs baseline: 5.3777x; 5.3777x over previous
import functools

import jax
import jax.numpy as jnp
from jax import lax
from jax.experimental import pallas as pl
from jax.experimental.pallas import tpu as pltpu
from jax.experimental.pallas import tpu_sc as plsc

N_NODES = 50000
N_EDGES = 800000
DIM = 100
DAGG = 50
DCLS = 30

ROW_BLK = 1024
NP = 50176
GRID = NP // ROW_BLK

NB = 4
CB = 25
CBP = 32

ND = 51200
TILE_ROWS = ND // 16
DUMMY_DST = 50000

EK = 128
EP = 802816
TILE_EDGES = EP // 16
N_BATCH = TILE_EDGES // EK


def _write_blocked(pre_ref, fp):
    r = fp.shape[0]
    pad = jnp.zeros((r, CBP - CB), jnp.float32)
    for j in range(NB):
        pre_ref[j] = jnp.concatenate([fp[:, j * CB:(j + 1) * CB], pad], axis=1)


def _init_body(feat_ref, wi_ref, bi_ref, w1_ref, b1_ref, w2_ref, b2_ref,
               h_ref, pre_ref):
    h = jnp.dot(feat_ref[...], wi_ref[...],
                preferred_element_type=jnp.float32) + bi_ref[...]
    h_ref[...] = h
    a = jnp.maximum(jnp.dot(h, w1_ref[...],
                            preferred_element_type=jnp.float32) + b1_ref[...], 0.0)
    fp = jnp.dot(a, w2_ref[...], preferred_element_type=jnp.float32) + b2_ref[...]
    _write_blocked(pre_ref, fp)


def _gru_common(msg_ref, h_ref, wir, wiz, win, whr, whz, whn,
                bir, biz, bin_, bhr, bhz, bhn):
    x = jnp.concatenate([msg_ref[j, :, :CB] for j in range(NB)], axis=1)
    h = h_ref[...]
    d = functools.partial(jnp.dot, preferred_element_type=jnp.float32)
    r = jax.nn.sigmoid(d(x, wir[...]) + bir[...] + d(h, whr[...]) + bhr[...])
    z = jax.nn.sigmoid(d(x, wiz[...]) + biz[...] + d(h, whz[...]) + bhz[...])
    n = jnp.tanh(d(x, win[...]) + bin_[...] + r * (d(h, whn[...]) + bhn[...]))
    return (1.0 - z) * n + z * h


def _gru_mlp_body(msg_ref, h_ref, wir, wiz, win, whr, whz, whn,
                  bir, biz, bin_, bhr, bhz, bhn,
                  w1_ref, b1_ref, w2_ref, b2_ref, hout_ref, pre_ref):
    hn = _gru_common(msg_ref, h_ref, wir, wiz, win, whr, whz, whn,
                     bir, biz, bin_, bhr, bhz, bhn)
    hout_ref[...] = hn
    a = jnp.maximum(jnp.dot(hn, w1_ref[...],
                            preferred_element_type=jnp.float32) + b1_ref[...], 0.0)
    fp = jnp.dot(a, w2_ref[...], preferred_element_type=jnp.float32) + b2_ref[...]
    _write_blocked(pre_ref, fp)


def _gru_cls_body(msg_ref, h_ref, wir, wiz, win, whr, whz, whn,
                  bir, biz, bin_, bhr, bhz, bhn,
                  w1_ref, b1_ref, w2_ref, b2_ref, out_ref):
    hn = _gru_common(msg_ref, h_ref, wir, wiz, win, whr, whz, whn,
                     bir, biz, bin_, bhr, bhz, bhn)
    a = jnp.maximum(jnp.dot(hn, w1_ref[...],
                            preferred_element_type=jnp.float32) + b1_ref[...], 0.0)
    out_ref[...] = jnp.dot(a, w2_ref[...],
                           preferred_element_type=jnp.float32) + b2_ref[...]


def _row_spec(cols):
    return pl.BlockSpec((ROW_BLK, cols), lambda i: (i, 0))


def _full_spec(shape):
    nd = len(shape)
    return pl.BlockSpec(shape, lambda i, _n=nd: (0,) * _n)


def _msg_spec():
    return pl.BlockSpec((NB, ROW_BLK, CBP), lambda i: (0, i, 0))


def _pre_out_spec():
    return pl.BlockSpec((NB, ROW_BLK, CBP), lambda i: (0, i, 0))


def _segsum_body(pre_hbm, gidx_hbm, sidx_hbm, out_hbm,
                 gidx_v0, gidx_v1, sidx_v0, sidx_v1, rows_v0, rows_v1,
                 acc_sh, sem0, sem1):
    c = lax.axis_index("c")
    s = lax.axis_index("s")
    row_base = s * TILE_ROWS
    edge_base = s * TILE_EDGES
    gidx = (gidx_v0, gidx_v1)
    sidx = (sidx_v0, sidx_v1)
    rows = (rows_v0, rows_v1)
    sems = (sem0, sem1)

    for p in range(2):
        blk = c * 2 + p

        def _zrow(i, _):
            rows_v0[i // 2, pl.ds((i % 2) * 16, 16)] = jnp.zeros((16,), jnp.float32)
            return _

        lax.fori_loop(0, EK * 2, _zrow, None)

        def _zacc(q, _):
            pltpu.sync_copy(rows_v0, acc_sh.at[pl.ds(row_base + q * EK, EK)])
            return _

        lax.fori_loop(0, TILE_ROWS // EK, _zacc, None)
        plsc.subcore_barrier()

        pltpu.sync_copy(gidx_hbm.at[pl.ds(edge_base, EK)], gidx[0])
        pltpu.sync_copy(sidx_hbm.at[pl.ds(edge_base, EK)], sidx[0])

        def _pair(g, _):
            for b in range(2):
                i = 2 * g + b
                oth = 1 - b
                hdl = pltpu.async_copy(
                    pre_hbm.at[blk].at[gidx[b]], rows[b], sems[b])

                @pl.when(i > 0)
                def _scatter_prev():
                    pltpu.sync_copy(rows[oth], acc_sh.at[sidx[oth]], add=True)

                @pl.when(i + 1 < N_BATCH)
                def _load_next():
                    pos = edge_base + (i + 1) * EK
                    pltpu.sync_copy(gidx_hbm.at[pl.ds(pos, EK)], gidx[oth])
                    pltpu.sync_copy(sidx_hbm.at[pl.ds(pos, EK)], sidx[oth])

                hdl.wait()
            return _

        lax.fori_loop(0, N_BATCH // 2, _pair, None)
        pltpu.sync_copy(rows[1], acc_sh.at[sidx[1]], add=True)
        plsc.subcore_barrier()

        pltpu.sync_copy(acc_sh.at[pl.ds(row_base, TILE_ROWS)],
                        out_hbm.at[blk].at[pl.ds(row_base, TILE_ROWS)])


_SEGSUM_CACHE = []


def _segsum_sc(pre, gidx, sidx):
    if not _SEGSUM_CACHE:
        mesh = plsc.VectorSubcoreMesh(core_axis_name="c", subcore_axis_name="s")
        _SEGSUM_CACHE.append(pl.kernel(
            _segsum_body,
            out_type=jax.ShapeDtypeStruct((NB, ND, CBP), jnp.float32),
            mesh=mesh,
            scratch_types=[
                pltpu.VMEM((EK,), jnp.int32),
                pltpu.VMEM((EK,), jnp.int32),
                pltpu.VMEM((EK,), jnp.int32),
                pltpu.VMEM((EK,), jnp.int32),
                pltpu.VMEM((EK, CBP), jnp.float32),
                pltpu.VMEM((EK, CBP), jnp.float32),
                pltpu.VMEM_SHARED((ND, CBP), jnp.float32),
                pltpu.SemaphoreType.DMA,
                pltpu.SemaphoreType.DMA,
            ],
            compiler_params=pltpu.CompilerParams(use_tc_tiling_on_sc=False),
        ))
    return _SEGSUM_CACHE[0](pre, gidx, sidx)


def _split3(w):
    return w[:, :DIM], w[:, DIM:2 * DIM], w[:, 2 * DIM:]


def kernel(features, edge_index, W_init, b_init, fm_W1, fm_b1, fm_W2, fm_b2,
           bm_W1, bm_b1, bm_W2, bm_b2, fg_Wih, fg_Whh, fg_bih, fg_bhh,
           bg_Wih, bg_Whh, bg_bih, bg_bhh, cl_W1, cl_b1, cl_W2, cl_b2):
    f32 = jnp.float32

    feat_p = jnp.concatenate(
        [features, jnp.zeros((NP - N_NODES, 4), f32)], axis=0)

    dst = edge_index[0]
    src = edge_index[1]
    pad_g = jnp.zeros((EP - N_EDGES,), jnp.int32)
    pad_s = jnp.full((EP - N_EDGES,), DUMMY_DST, jnp.int32)
    src_g = jnp.concatenate([src, pad_g])
    dst_s = jnp.concatenate([dst, pad_s])
    dst_g = jnp.concatenate([dst, pad_g])
    src_s = jnp.concatenate([src, pad_s])

    fir, fiz, fin = _split3(fg_Wih)
    fhr, fhz, fhn = _split3(fg_Whh)
    bir_, biz_, bin_ = _split3(bg_Wih)
    bhr_, bhz_, bhn_ = _split3(bg_Whh)

    def b2d(b):
        return b.reshape(1, -1)

    fb = [b2d(x) for x in jnp.split(fg_bih, 3)] + [b2d(x) for x in jnp.split(fg_bhh, 3)]
    bb = [b2d(x) for x in jnp.split(bg_bih, 3)] + [b2d(x) for x in jnp.split(bg_bhh, 3)]

    h_shape = jax.ShapeDtypeStruct((NP, DIM), f32)
    pre_shape = jax.ShapeDtypeStruct((NB, NP, CBP), f32)

    h, fpre = pl.pallas_call(
        _init_body,
        grid=(GRID,),
        in_specs=[
            _row_spec(4),
            _full_spec((4, DIM)), _full_spec((1, DIM)),
            _full_spec((DIM, DAGG)), _full_spec((1, DAGG)),
            _full_spec((DAGG, DIM)), _full_spec((1, DIM)),
        ],
        out_specs=[_row_spec(DIM), _pre_out_spec()],
        out_shape=[h_shape, pre_shape],
    )(feat_p, W_init, b2d(b_init), fm_W1, b2d(fm_b1), fm_W2, b2d(fm_b2))

    gru_mlp_specs = dict(
        grid=(GRID,),
        in_specs=[
            _msg_spec(), _row_spec(DIM),
            _full_spec((DIM, DIM)), _full_spec((DIM, DIM)), _full_spec((DIM, DIM)),
            _full_spec((DIM, DIM)), _full_spec((DIM, DIM)), _full_spec((DIM, DIM)),
            _full_spec((1, DIM)), _full_spec((1, DIM)), _full_spec((1, DIM)),
            _full_spec((1, DIM)), _full_spec((1, DIM)), _full_spec((1, DIM)),
            _full_spec((DIM, DAGG)), _full_spec((1, DAGG)),
            _full_spec((DAGG, DIM)), _full_spec((1, DIM)),
        ],
    )

    gru_mlp = pl.pallas_call(
        _gru_mlp_body,
        out_specs=[_row_spec(DIM), _pre_out_spec()],
        out_shape=[h_shape, pre_shape],
        **gru_mlp_specs,
    )

    cls_specs = dict(gru_mlp_specs)
    cls_specs["in_specs"] = cls_specs["in_specs"][:-4] + [
        _full_spec((DIM, DCLS)), _full_spec((1, DCLS)),
        _full_spec((DCLS, 1)), _full_spec((1, 1)),
    ]
    gru_cls = pl.pallas_call(
        _gru_cls_body,
        out_specs=[_row_spec(1)],
        out_shape=[jax.ShapeDtypeStruct((NP, 1), f32)],
        **cls_specs,
    )

    for rnd in range(4):
        fmsg = _segsum_sc(fpre, src_g, dst_s)
        h, bpre = gru_mlp(
            fmsg, h, fir, fiz, fin, fhr, fhz, fhn, *fb,
            bm_W1, b2d(bm_b1), bm_W2, b2d(bm_b2))
        bmsg = _segsum_sc(bpre, dst_g, src_s)
        if rnd < 3:
            h, fpre = gru_mlp(
                bmsg, h, bir_, biz_, bin_, bhr_, bhz_, bhn_, *bb,
                fm_W1, b2d(fm_b1), fm_W2, b2d(fm_b2))
        else:
            (out,) = gru_cls(
                bmsg, h, bir_, biz_, bin_, bhr_, bhz_, bhn_, *bb,
                cl_W1, b2d(cl_b1), cl_W2, b2d(cl_b2))

    return out[:N_NODES]
